# baseline (speedup 1.0000x reference)
import functools

import jax
import jax.numpy as jnp
from jax import lax
from jax.experimental import pallas as pl
from jax.experimental.pallas import tpu as pltpu
from jax.experimental.pallas import tpu_sc as plsc

NUM_EMB = 1024
DIM = 256
N_TOK = 16 * 1024
CCOST = 0.25

NC = 2
NS = 16
NW = NC * NS
RW = N_TOK // NW
CH = 128
NCH = RW // CH


def _vq_tc_body(x2_ref, x_ref, w_ref, w2_ref, lane_ref,
                idx_ref, loss_ref):
    i = pl.program_id(0)
    g = pl.num_programs(0)
    x = x_ref[...]
    w = w_ref[...]
    lane = lane_ref[...]

    scores = lax.dot_general(
        x, w, (((1,), (1,)), ((), ())),
        preferred_element_type=jnp.float32)
    t = x2_ref[...] + w2_ref[...]
    d = t - 2.0 * scores

    m = jnp.min(d, axis=1, keepdims=True)
    idxf = jnp.min(jnp.where(d == m, lane, jnp.float32(NUM_EMB)),
                   axis=1, keepdims=True)
    idx_ref[...] = idxf.astype(jnp.int32).reshape(1, 1, -1)

    @pl.when(i == 0)
    def _init():
        loss_ref[...] = jnp.zeros_like(loss_ref)

    loss_ref[...] += jnp.sum(m).reshape(1, 1)

    @pl.when(i == g - 1)
    def _finalize():
        s = loss_ref[...] / jnp.float32(N_TOK * DIM)
        loss_ref[...] = s + CCOST * s


@functools.partial(jax.jit, static_argnames=("block",))
def _vq_tc(flat, x2, w, w2, lane, block=2048):
    g = N_TOK // block
    return pl.pallas_call(
        _vq_tc_body,
        grid=(g,),
        in_specs=[
            pl.BlockSpec((block, 1), lambda i: (i, 0)),
            pl.BlockSpec((block, DIM), lambda i: (i, 0)),
            pl.BlockSpec((NUM_EMB, DIM), lambda i: (0, 0)),
            pl.BlockSpec((1, NUM_EMB), lambda i: (0, 0)),
            pl.BlockSpec((1, NUM_EMB), lambda i: (0, 0)),
        ],
        out_specs=[
            pl.BlockSpec((1, 1, block), lambda i: (i, 0, 0)),
            pl.BlockSpec((1, 1), lambda i: (0, 0)),
        ],
        out_shape=[
            jax.ShapeDtypeStruct((g, 1, block), jnp.int32),
            jax.ShapeDtypeStruct((1, 1), jnp.float32),
        ],
        compiler_params=pltpu.CompilerParams(
            dimension_semantics=("arbitrary",)),
    )(x2, flat, w, w2, lane)


def _vq_sc_body(w_hbm, idx_hbm, qst_hbm, cnt_hbm,
                idxv, rowa, rowb, cntv, sema, semb):
    c = lax.axis_index("c")
    s = lax.axis_index("s")
    wid = s * NC + c
    base = wid * RW

    pltpu.sync_copy(idx_hbm.at[wid], idxv)

    for j in range(NUM_EMB // 16):
        cntv[pl.ds(j * 16, 16)] = jnp.zeros((16,), jnp.float32)

    cp0 = pltpu.async_copy(w_hbm.at[idxv.at[0]], rowa, sema)
    cp1 = pltpu.async_copy(w_hbm.at[idxv.at[1]], rowb, semb)
    cp0.wait()
    pltpu.sync_copy(rowa, qst_hbm.at[pl.ds(base, CH)])
    cp2 = pltpu.async_copy(w_hbm.at[idxv.at[2]], rowa, sema)
    cp1.wait()
    pltpu.sync_copy(rowb, qst_hbm.at[pl.ds(base + CH, CH)])
    cp3 = pltpu.async_copy(w_hbm.at[idxv.at[3]], rowb, semb)
    cp2.wait()
    pltpu.sync_copy(rowa, qst_hbm.at[pl.ds(base + 2 * CH, CH)])
    cp3.wait()
    pltpu.sync_copy(rowb, qst_hbm.at[pl.ds(base + 3 * CH, CH)])

    ones = jnp.ones((16,), jnp.float32)
    for ch in range(NCH):
        for j in range(CH // 16):
            iv = idxv[ch, pl.ds(j * 16, 16)]
            plsc.addupdate_scatter(cntv, [iv], ones)
    pltpu.sync_copy(cntv, cnt_hbm.at[wid])


@jax.jit
def _vq_sc(w, idx3):
    mesh = plsc.VectorSubcoreMesh(core_axis_name="c", subcore_axis_name="s")
    f = pl.kernel(
        _vq_sc_body,
        mesh=mesh,
        out_type=[
            jax.ShapeDtypeStruct((N_TOK, DIM), jnp.float32),
            jax.ShapeDtypeStruct((NW, NUM_EMB), jnp.float32),
        ],
        scratch_types=[
            pltpu.VMEM((NCH, CH), jnp.int32),
            pltpu.VMEM((CH, DIM), jnp.float32),
            pltpu.VMEM((CH, DIM), jnp.float32),
            pltpu.VMEM((NUM_EMB,), jnp.float32),
            pltpu.SemaphoreType.DMA,
            pltpu.SemaphoreType.DMA,
        ],
        compiler_params=pltpu.CompilerParams(needs_layout_passes=False),
    )
    return f(w, idx3)


def kernel(inputs, embedding_weight):
    input_shape = inputs.shape
    flat = inputs.reshape(-1, DIM)
    x2 = jnp.sum(flat ** 2, axis=1, keepdims=True)
    w2 = jnp.sum(embedding_weight ** 2, axis=1)[None, :]
    lane = lax.broadcasted_iota(jnp.float32, (1, NUM_EMB), 1)
    idx, loss = _vq_tc(flat, x2, embedding_weight, w2, lane)
    idx3 = idx.reshape(NW, NCH, CH)
    qst, cnt_part = _vq_sc(embedding_weight, idx3)
    p = jnp.sum(cnt_part, axis=0) / jnp.float32(N_TOK)
    perp = jnp.exp(-jnp.sum(p * jnp.log(p + 1e-10)))
    return (qst.reshape(input_shape),
            loss.reshape(()),
            perp,
            idx.reshape(input_shape[:-1]))


# speedup vs baseline: 1.5232x; 1.5232x over previous
import functools

import jax
import jax.numpy as jnp
from jax import lax
from jax.experimental import pallas as pl
from jax.experimental.pallas import tpu as pltpu

NUM_EMB = 1024
DIM = 256
N_TOK = 16 * 1024
CCOST = 0.25


def _vq_body(x_ref, w_ref, w2_ref,
             qst_ref, idx_ref, counts_ref, loss_ref, perp_ref):
    i = pl.program_id(0)
    g = pl.num_programs(0)
    x = x_ref[...]
    w = w_ref[...]
    lane = lax.broadcasted_iota(jnp.int32, (1, NUM_EMB), 1).astype(jnp.float32)

    scores = lax.dot_general(
        x, w, (((1,), (1,)), ((), ())),
        preferred_element_type=jnp.float32)
    x2 = jnp.sum(x * x, axis=1, keepdims=True)
    t = x2 + w2_ref[...]
    d = t - 2.0 * scores

    m = jnp.min(d, axis=1, keepdims=True)
    idxf = jnp.min(jnp.where(d == m, lane, jnp.float32(NUM_EMB)),
                   axis=1, keepdims=True)
    idx_ref[...] = idxf.astype(jnp.int32).reshape(1, 1, -1)

    onehot = (lane == idxf).astype(jnp.float32)
    q = lax.dot_general(
        onehot, w, (((1,), (0,)), ((), ())),
        preferred_element_type=jnp.float32)
    qst_ref[...] = x + (q - x)

    @pl.when(i == 0)
    def _init():
        counts_ref[...] = jnp.zeros_like(counts_ref)
        loss_ref[...] = jnp.zeros_like(loss_ref)

    counts_ref[...] += jnp.sum(onehot, axis=0, keepdims=True)
    loss_ref[...] += jnp.sum(m).reshape(1, 1)

    @pl.when(i == g - 1)
    def _finalize():
        s = loss_ref[...] / jnp.float32(N_TOK * DIM)
        loss_ref[...] = s + CCOST * s
        p = counts_ref[...] / jnp.float32(N_TOK)
        perp_ref[...] = jnp.exp(-jnp.sum(p * jnp.log(p + 1e-10))).reshape(1, 1)


@functools.partial(jax.jit, static_argnames=("block",))
def _vq_tc(flat, w, w2, block=2048):
    g = N_TOK // block
    out = pl.pallas_call(
        _vq_body,
        grid=(g,),
        in_specs=[
            pl.BlockSpec((block, DIM), lambda i: (i, 0)),
            pl.BlockSpec((NUM_EMB, DIM), lambda i: (0, 0)),
            pl.BlockSpec((1, NUM_EMB), lambda i: (0, 0)),
        ],
        out_specs=[
            pl.BlockSpec((block, DIM), lambda i: (i, 0)),
            pl.BlockSpec((1, 1, block), lambda i: (i, 0, 0)),
            pl.BlockSpec((1, NUM_EMB), lambda i: (0, 0)),
            pl.BlockSpec((1, 1), lambda i: (0, 0)),
            pl.BlockSpec((1, 1), lambda i: (0, 0)),
        ],
        out_shape=[
            jax.ShapeDtypeStruct((N_TOK, DIM), jnp.float32),
            jax.ShapeDtypeStruct((g, 1, block), jnp.int32),
            jax.ShapeDtypeStruct((1, NUM_EMB), jnp.float32),
            jax.ShapeDtypeStruct((1, 1), jnp.float32),
            jax.ShapeDtypeStruct((1, 1), jnp.float32),
        ],
        compiler_params=pltpu.CompilerParams(
            dimension_semantics=("arbitrary",)),
    )(flat, w, w2)
    return out


def kernel(inputs, embedding_weight):
    input_shape = inputs.shape
    flat = inputs.reshape(-1, DIM)
    w2 = jnp.sum(embedding_weight ** 2, axis=1)[None, :]
    qst, idx, _counts, loss, perp = _vq_tc(flat, embedding_weight, w2)
    return (qst.reshape(input_shape),
            loss.reshape(()),
            perp.reshape(()),
            idx.reshape(input_shape[:-1]))


# speedup vs baseline: 1.5394x; 1.0107x over previous
import functools

import jax
import jax.numpy as jnp
from jax import lax
from jax.experimental import pallas as pl
from jax.experimental.pallas import tpu as pltpu

NUM_EMB = 1024
DIM = 256
N_TOK = 16 * 1024
CCOST = 0.25


def _vq_body(x_ref, w_ref, wdbl_ref, w2_ref,
             qst_ref, idx_ref, counts_ref, loss_ref, perp_ref):
    i = pl.program_id(0)
    g = pl.num_programs(0)
    x = x_ref[...]
    w = w_ref[...]
    lane = lax.broadcasted_iota(jnp.int32, (1, NUM_EMB), 1).astype(jnp.float32)

    scores2 = lax.dot_general(
        x, wdbl_ref[...], (((1,), (1,)), ((), ())),
        preferred_element_type=jnp.float32)
    x2 = jnp.sum(x * x, axis=1, keepdims=True)
    t = x2 + w2_ref[...]
    d = t - scores2

    m = jnp.min(d, axis=1, keepdims=True)
    idxf = jnp.min(jnp.where(d == m, lane, jnp.float32(NUM_EMB)),
                   axis=1, keepdims=True)
    idx_ref[...] = idxf.astype(jnp.int32).reshape(1, 1, -1)

    onehot = (lane == idxf).astype(jnp.float32)
    q = lax.dot_general(
        onehot, w, (((1,), (0,)), ((), ())),
        preferred_element_type=jnp.float32)
    qst_ref[...] = q

    @pl.when(i == 0)
    def _init():
        counts_ref[...] = jnp.zeros_like(counts_ref)
        loss_ref[...] = jnp.zeros_like(loss_ref)

    ones_row = jnp.ones((1, onehot.shape[0]), jnp.float32)
    counts_ref[...] += lax.dot_general(
        ones_row, onehot, (((1,), (0,)), ((), ())),
        preferred_element_type=jnp.float32)
    loss_ref[...] += jnp.sum(m).reshape(1, 1)

    @pl.when(i == g - 1)
    def _finalize():
        s = loss_ref[...] / jnp.float32(N_TOK * DIM)
        loss_ref[...] = s + CCOST * s
        p = counts_ref[...] / jnp.float32(N_TOK)
        perp_ref[...] = jnp.exp(-jnp.sum(p * jnp.log(p + 1e-10))).reshape(1, 1)


@functools.partial(jax.jit, static_argnames=("block",))
def _vq_tc(flat, w, wdbl, w2, block=2048):
    g = N_TOK // block
    out = pl.pallas_call(
        _vq_body,
        grid=(g,),
        in_specs=[
            pl.BlockSpec((block, DIM), lambda i: (i, 0)),
            pl.BlockSpec((NUM_EMB, DIM), lambda i: (0, 0)),
            pl.BlockSpec((NUM_EMB, DIM), lambda i: (0, 0)),
            pl.BlockSpec((1, NUM_EMB), lambda i: (0, 0)),
        ],
        out_specs=[
            pl.BlockSpec((block, DIM), lambda i: (i, 0)),
            pl.BlockSpec((1, 1, block), lambda i: (i, 0, 0)),
            pl.BlockSpec((1, NUM_EMB), lambda i: (0, 0)),
            pl.BlockSpec((1, 1), lambda i: (0, 0)),
            pl.BlockSpec((1, 1), lambda i: (0, 0)),
        ],
        out_shape=[
            jax.ShapeDtypeStruct((N_TOK, DIM), jnp.float32),
            jax.ShapeDtypeStruct((g, 1, block), jnp.int32),
            jax.ShapeDtypeStruct((1, NUM_EMB), jnp.float32),
            jax.ShapeDtypeStruct((1, 1), jnp.float32),
            jax.ShapeDtypeStruct((1, 1), jnp.float32),
        ],
        compiler_params=pltpu.CompilerParams(
            dimension_semantics=("arbitrary",)),
    )(flat, w, wdbl, w2)
    return out


def kernel(inputs, embedding_weight):
    input_shape = inputs.shape
    flat = inputs.reshape(-1, DIM)
    w2 = jnp.sum(embedding_weight ** 2, axis=1)[None, :]
    wdbl = embedding_weight + embedding_weight
    qst, idx, _counts, loss, perp = _vq_tc(flat, embedding_weight, wdbl, w2)
    return (qst.reshape(input_shape),
            loss.reshape(()),
            perp.reshape(()),
            idx.reshape(input_shape[:-1]))


# speedup vs baseline: 1.5636x; 1.0157x over previous
import functools

import jax
import jax.numpy as jnp
from jax import lax
from jax.experimental import pallas as pl
from jax.experimental.pallas import tpu as pltpu

NUM_EMB = 1024
DIM = 256
N_TOK = 16 * 1024
CCOST = 0.25


def _vq_body(x_ref, w_ref, wdbl_ref, w2_ref,
             qst_ref, idx_ref, counts_ref, loss_ref, perp_ref):
    i = pl.program_id(0)
    g = pl.num_programs(0)
    x = x_ref[...]
    w = w_ref[...]
    lane = lax.broadcasted_iota(jnp.int32, (1, NUM_EMB), 1).astype(jnp.float32)

    scores2 = lax.dot_general(
        x, wdbl_ref[...], (((1,), (1,)), ((), ())),
        preferred_element_type=jnp.float32)
    x2 = jnp.sum(x * x, axis=1, keepdims=True)
    t = x2 + w2_ref[...]
    d = t - scores2

    m = jnp.min(d, axis=1, keepdims=True)
    idxf = jnp.min(jnp.where(d == m, lane, jnp.float32(NUM_EMB)),
                   axis=1, keepdims=True)
    idx_ref[...] = idxf.astype(jnp.int32).reshape(1, 1, -1)

    onehot = (lane == idxf).astype(jnp.float32)
    q = lax.dot_general(
        onehot, w, (((1,), (0,)), ((), ())),
        preferred_element_type=jnp.float32)
    qst_ref[...] = q

    @pl.when(i == 0)
    def _init():
        counts_ref[...] = jnp.zeros_like(counts_ref)
        loss_ref[...] = jnp.zeros_like(loss_ref)

    ones_row = jnp.ones((1, onehot.shape[0]), jnp.float32)
    counts_ref[...] += lax.dot_general(
        ones_row, onehot, (((1,), (0,)), ((), ())),
        preferred_element_type=jnp.float32)
    loss_ref[...] += jnp.sum(m).reshape(1, 1)

    @pl.when(i == g - 1)
    def _finalize():
        s = loss_ref[...] / jnp.float32(N_TOK * DIM)
        loss_ref[...] = s + CCOST * s
        p = counts_ref[...] / jnp.float32(N_TOK)
        perp_ref[...] = jnp.exp(-jnp.sum(p * jnp.log(p + 1e-10))).reshape(1, 1)


@functools.partial(jax.jit, static_argnames=("block",))
def _vq_tc(flat, w, wdbl, w2, block=4096):
    g = N_TOK // block
    out = pl.pallas_call(
        _vq_body,
        grid=(g,),
        in_specs=[
            pl.BlockSpec((block, DIM), lambda i: (i, 0)),
            pl.BlockSpec((NUM_EMB, DIM), lambda i: (0, 0)),
            pl.BlockSpec((NUM_EMB, DIM), lambda i: (0, 0)),
            pl.BlockSpec((1, NUM_EMB), lambda i: (0, 0)),
        ],
        out_specs=[
            pl.BlockSpec((block, DIM), lambda i: (i, 0)),
            pl.BlockSpec((1, 1, block), lambda i: (i, 0, 0)),
            pl.BlockSpec((1, NUM_EMB), lambda i: (0, 0)),
            pl.BlockSpec((1, 1), lambda i: (0, 0)),
            pl.BlockSpec((1, 1), lambda i: (0, 0)),
        ],
        out_shape=[
            jax.ShapeDtypeStruct((N_TOK, DIM), jnp.float32),
            jax.ShapeDtypeStruct((g, 1, block), jnp.int32),
            jax.ShapeDtypeStruct((1, NUM_EMB), jnp.float32),
            jax.ShapeDtypeStruct((1, 1), jnp.float32),
            jax.ShapeDtypeStruct((1, 1), jnp.float32),
        ],
        compiler_params=pltpu.CompilerParams(
            dimension_semantics=("arbitrary",)),
    )(flat, w, wdbl, w2)
    return out


def kernel(inputs, embedding_weight):
    input_shape = inputs.shape
    flat = inputs.reshape(-1, DIM)
    w2 = jnp.sum(embedding_weight ** 2, axis=1)[None, :]
    wdbl = embedding_weight + embedding_weight
    qst, idx, _counts, loss, perp = _vq_tc(flat, embedding_weight, wdbl, w2)
    return (qst.reshape(input_shape),
            loss.reshape(()),
            perp.reshape(()),
            idx.reshape(input_shape[:-1]))
